# baseline (speedup 1.0000x reference)
import jax
import jax.numpy as jnp
from jax import lax
from jax.experimental import pallas as pl
from jax.experimental.pallas import tpu as pltpu
from jax.experimental.pallas import tpu_sc as plsc

M = 16
MAX_NORM = 1.0 - 1e-05
MAX_NORM_SQ = MAX_NORM * MAX_NORM

NC = 2
NS = 16
NW = NC * NS
L = 16

B_TOTAL = 16384 * 26
B_PER_W = B_TOTAL // NW
NCHUNK = 4
C = B_PER_W // NCHUNK
GROUPS = C // L


def _rsqrt16(ss):
    bits = lax.bitcast_convert_type(ss, jnp.int32)
    y = lax.bitcast_convert_type(
        jnp.int32(0x5F3759DF) - lax.shift_right_arithmetic(bits, 1),
        jnp.float32)
    for _ in range(3):
        y = y * (1.5 - 0.5 * ss * y * y)
    return y


def _normalize_chunk(buf):
    iota = lax.iota(jnp.int32, L)

    def body(g, carry):
        row_ids = g * L + iota
        cols = [
            plsc.load_gather(buf, [row_ids, jnp.full((L,), j, jnp.int32)])
            for j in range(M)
        ]
        ss = cols[0] * cols[0]
        for j in range(1, M):
            ss = ss + cols[j] * cols[j]
        scale = jnp.where(ss > MAX_NORM_SQ, MAX_NORM * _rsqrt16(ss),
                          jnp.float32(1.0))
        for j in range(M):
            plsc.store_scatter(buf, [row_ids, jnp.full((L,), j, jnp.int32)],
                               cols[j] * scale)
        return carry

    lax.fori_loop(0, GROUPS, body, 0, unroll=False)


def _sc_kernel(w_hbm, idx_hbm, out_hbm, idx_v, rows_a, rows_b, sem_a, sem_b):
    wid = lax.axis_index("s") * NC + lax.axis_index("c")
    base = wid * B_PER_W

    pltpu.sync_copy(idx_hbm.at[pl.ds(base, B_PER_W)], idx_v)

    bufs = (rows_a, rows_b)
    sems = (sem_a, sem_b)

    def gather(c, b):
        return pltpu.async_copy(
            w_hbm.at[idx_v.at[pl.ds(c * C, C)]], bufs[b], sems[b])

    handles = [gather(0, 0), None]
    for c in range(NCHUNK):
        b = c % 2
        if c + 1 < NCHUNK:
            handles[(c + 1) % 2] = gather(c + 1, (c + 1) % 2)
        handles[b].wait()
        _normalize_chunk(bufs[b])
        pltpu.sync_copy(bufs[b], out_hbm.at[pl.ds(base + c * C, C)])


@jax.jit
def _run(idx_flat, weight):
    mesh = plsc.VectorSubcoreMesh(core_axis_name="c", subcore_axis_name="s")
    f = pl.kernel(
        _sc_kernel,
        out_type=jax.ShapeDtypeStruct((B_TOTAL, M), jnp.float32),
        mesh=mesh,
        compiler_params=pltpu.CompilerParams(
            needs_layout_passes=False, use_tc_tiling_on_sc=False),
        scratch_types=[
            pltpu.VMEM((B_PER_W,), jnp.int32),
            pltpu.VMEM((C, M), jnp.float32),
            pltpu.VMEM((C, M), jnp.float32),
            pltpu.SemaphoreType.DMA,
            pltpu.SemaphoreType.DMA,
        ],
    )
    return f(weight, idx_flat)


def kernel(x, weight):
    idx_flat = x.reshape(-1).astype(jnp.int32)
    out = _run(idx_flat, weight)
    return out.reshape(x.shape + (M,))


# speedup vs baseline: 1.8788x; 1.8788x over previous
import jax
import jax.numpy as jnp
from jax import lax
from jax.experimental import pallas as pl
from jax.experimental.pallas import tpu as pltpu
from jax.experimental.pallas import tpu_sc as plsc

M = 16
MAX_NORM = 1.0 - 1e-05
MAX_NORM_SQ = MAX_NORM * MAX_NORM

NC = 2
NS = 16
NW = NC * NS
L = 16

B = 16384
F = 26
B_TOTAL = B * F
B_PER_W = B_TOTAL // NW
BLK = 128
NBLK_W = B_PER_W // BLK
NSC = 8
BLK_SC = NBLK_W // NSC
C = BLK_SC * BLK
GROUPS = BLK // L


def _rsqrt16(ss):
    bits = lax.bitcast_convert_type(ss, jnp.int32)
    y = lax.bitcast_convert_type(
        jnp.int32(0x5F3759DF) - lax.shift_right_arithmetic(bits, 1),
        jnp.float32)
    for _ in range(3):
        y = y * (1.5 - 0.5 * ss * y * y)
    return y


def _sc_kernel(w_hbm, idx_hbm, out_hbm, idx_v, rows_a, rows_b, outt,
               sem_a, sem_b, sem_oa, sem_ob):
    wid = lax.axis_index("s") * NC + lax.axis_index("c")
    base = wid * B_PER_W
    iota = lax.iota(jnp.int32, L)

    pltpu.sync_copy(idx_hbm.at[pl.ds(base, B_PER_W)], idx_v)

    rows = (rows_a, rows_b)
    gsems = (sem_a, sem_b)
    osems = (sem_oa, sem_ob)

    def gather(sc):
        b = sc % 2
        return pltpu.async_copy(
            w_hbm.at[idx_v.at[pl.ds(sc * C, C)]], rows[b], gsems[b])

    def compute_sc(sc):
        cur = sc % 2
        buf = rows[cur]

        def body(blk, carry):
            for g in range(GROUPS):
                row_ids = blk * BLK + g * L + iota
                cols = [
                    plsc.load_gather(
                        buf, [row_ids, jnp.full((L,), j, jnp.int32)])
                    for j in range(M)
                ]
                ss = cols[0] * cols[0]
                for j in range(1, M):
                    ss = ss + cols[j] * cols[j]
                scale = jnp.where(ss > MAX_NORM_SQ, MAX_NORM * _rsqrt16(ss),
                                  jnp.float32(1.0))
                for j in range(M):
                    outt[cur, j // 8, blk,
                         pl.ds((j % 8) * BLK + g * L, L)] = cols[j] * scale
            return carry

        lax.fori_loop(0, BLK_SC, body, 0, unroll=False)

    def emit_out(sc):
        cur = sc % 2
        handles = []
        for blk in range(BLK_SC):
            gblk = wid * NBLK_W + sc * BLK_SC + blk
            f26 = gblk // (B // BLK)
            cpos = gblk % (B // BLK)
            for band in range(2):
                off = (f26 * 256 + band * (B // BLK) + cpos) * 1024
                off = pl.multiple_of(off, 1024)
                handles.append(pltpu.async_copy(
                    outt.at[cur, band, blk],
                    out_hbm.at[pl.ds(off, 1024)], osems[cur]))
        return handles

    pending_out = [None, None]
    ghandles = {0: gather(0)}
    for sc in range(NSC):
        b = sc % 2
        if sc + 1 < NSC:
            ghandles[sc + 1] = gather(sc + 1)
        ghandles.pop(sc).wait()
        if pending_out[b] is not None:
            for h in pending_out[b]:
                h.wait()
        compute_sc(sc)
        pending_out[b] = emit_out(sc)
    for par in pending_out:
        if par is not None:
            for h in par:
                h.wait()


@jax.jit
def _run(idx_flat, weight):
    mesh = plsc.VectorSubcoreMesh(core_axis_name="c", subcore_axis_name="s")
    f = pl.kernel(
        _sc_kernel,
        out_type=jax.ShapeDtypeStruct((F * 2 * (B // BLK) * 1024,),
                                      jnp.float32),
        mesh=mesh,
        compiler_params=pltpu.CompilerParams(
            needs_layout_passes=False, use_tc_tiling_on_sc=False),
        scratch_types=[
            pltpu.VMEM((B_PER_W,), jnp.int32),
            pltpu.VMEM((C, M), jnp.float32),
            pltpu.VMEM((C, M), jnp.float32),
            pltpu.VMEM((2, 2, BLK_SC, 1024), jnp.float32),
            pltpu.SemaphoreType.DMA,
            pltpu.SemaphoreType.DMA,
            pltpu.SemaphoreType.DMA,
            pltpu.SemaphoreType.DMA,
        ],
    )
    return f(weight, idx_flat)


def kernel(x, weight):
    idx_flat = x.T.reshape(-1).astype(jnp.int32)
    out_flat = _run(idx_flat, weight)
    out5 = out_flat.reshape(F, 2, B // BLK, 8, BLK)
    return out5.transpose(2, 4, 0, 1, 3).reshape(B, F, M)
